# baseline (speedup 1.0000x reference)
import functools

import jax
import jax.numpy as jnp
from jax import lax
from jax.experimental import pallas as pl
from jax.experimental.pallas import tpu as pltpu
from jax.experimental.pallas import tpu_sc as plsc

G0 = 256
G1 = 256
D = 256
NUM_WORKERS = 32
ROWS_PER_WORKER = G0 // NUM_WORKERS
S_ROWS = 2 * G0 - 1


W_ROWS = G1 + ROWS_PER_WORKER - 1
LANES = 16


def _sc_body(bl_hbm, tr_hbm, out_hbm, vecs_v, s_v, sem):
    pltpu.sync_copy(bl_hbm, vecs_v.at[pl.ds(0, D)])
    pltpu.sync_copy(tr_hbm, vecs_v.at[pl.ds(D, D)])
    bl = [vecs_v[pl.ds(k * LANES, LANES)] for k in range(D // LANES)]
    tr = [vecs_v[pl.ds(D + k * LANES, LANES)] for k in range(D // LANES)]
    wid = lax.axis_index("s") * 2 + lax.axis_index("c")
    base = wid * ROWS_PER_WORKER
    n_bl = jnp.clip(G0 - 1 - base, 0, W_ROWS)

    def fill_bl(r, _):
        for k in range(D // LANES):
            s_v[pl.ds(r * D + k * LANES, LANES)] = bl[k]
        return _

    def fill_tr(r, _):
        for k in range(D // LANES):
            s_v[pl.ds(r * D + k * LANES, LANES)] = tr[k]
        return _

    lax.fori_loop(0, n_bl, fill_bl, None)
    lax.fori_loop(n_bl, W_ROWS, fill_tr, None)
    copies = [
        pltpu.async_copy(s_v.at[pl.ds(j * D, G1 * D)], out_hbm.at[base + j], sem)
        for j in range(ROWS_PER_WORKER)
    ]
    for c in copies:
        c.wait()


_sc_fill = functools.partial(
    pl.kernel,
    out_type=jax.ShapeDtypeStruct((G0, G1 * D), jnp.float32),
    mesh=plsc.VectorSubcoreMesh(core_axis_name="c", subcore_axis_name="s"),
    scratch_types=[
        pltpu.VMEM((2 * D,), jnp.float32),
        pltpu.VMEM((W_ROWS * D,), jnp.float32),
        pltpu.SemaphoreType.DMA,
    ],
)(_sc_body)


def kernel(left_to_right, right_to_left, top_to_bottom, bottom_to_top,
           top_left_to_bottom_right, bottom_right_to_top_left,
           bottom_left_to_top_right, top_right_to_bottom_left):
    out = _sc_fill(bottom_left_to_top_right, top_right_to_bottom_left)
    return out.reshape(G0, G1, D)
